# baseline (speedup 1.0000x reference)
import functools

import jax
import jax.numpy as jnp
from jax import lax
from jax.experimental import pallas as pl
from jax.experimental.pallas import tpu as pltpu
from jax.experimental.pallas import tpu_sc as plsc

ROWS = 64
COLS = 32768
N_TOTAL = ROWS * COLS
NC = 2
NS = 16
L = 16
NW = NC * NS

SC_ROWS = 16
TC_ROWS = ROWS - SC_ROWS
SC_PER_W = SC_ROWS * COLS // NW
CHUNK = 8192
NCHK = SC_PER_W // CHUNK
U = 8

_mesh = plsc.VectorSubcoreMesh(core_axis_name="c", subcore_axis_name="s")


@functools.partial(
    pl.kernel,
    mesh=_mesh,
    out_type=jax.ShapeDtypeStruct((NW, 3 * L), jnp.float32),
    scratch_types=[
        pltpu.VMEM((2, CHUNK), jnp.float32),
        pltpu.VMEM((2, CHUNK), jnp.float32),
        pltpu.VMEM((3 * L,), jnp.float32),
        pltpu.SemaphoreType.DMA,
        pltpu.SemaphoreType.DMA,
        pltpu.SemaphoreType.DMA,
        pltpu.SemaphoreType.DMA,
    ],
)
def _sc_partial(x_hbm, t_hbm, out_hbm, xv, tv, outv, sx0, sx1, st0, st1):
    wid = lax.axis_index("s") * NC + lax.axis_index("c")
    elem0 = wid * SC_PER_W
    zero = jnp.zeros((L,), jnp.float32)
    xsems = (sx0, sx1)
    tsems = (st0, st1)

    def start(i):
        e = elem0 + i * CHUNK
        r = e // COLS
        off = e % COLS
        b = i % 2
        hx = pltpu.async_copy(x_hbm.at[r, pl.ds(off, CHUNK)], xv.at[b], xsems[b])
        ht = pltpu.async_copy(t_hbm.at[r, pl.ds(off, CHUNK)], tv.at[b], tsems[b])
        return (hx, ht)

    accs = [zero] * (3 * U)
    handles = {0: start(0)}
    for i in range(NCHK):
        if i + 1 < NCHK:
            handles[i + 1] = start(i + 1)
        hx, ht = handles.pop(i)
        hx.wait()
        ht.wait()
        b = i % 2

        def vec_body(j, acc, _b=b):
            acc = list(acc)
            base = j * (U * L)
            for k in range(U):
                x = xv[_b, pl.ds(base + k * L, L)]
                t = tv[_b, pl.ds(base + k * L, L)]
                d = x - t
                sq = d * d
                acc[k] = acc[k] + sq * t
                acc[U + k] = acc[U + k] + sq
                acc[2 * U + k] = acc[2 * U + k] + t
            return tuple(acc)

        accs = lax.fori_loop(0, CHUNK // (U * L), vec_body, tuple(accs))

    a_bt = functools.reduce(lambda a, b: a + b, accs[0:U])
    a_sq = functools.reduce(lambda a, b: a + b, accs[U:2 * U])
    a_ct = functools.reduce(lambda a, b: a + b, accs[2 * U:3 * U])
    outv[pl.ds(0, L)] = a_bt
    outv[pl.ds(L, L)] = a_sq
    outv[pl.ds(2 * L, L)] = a_ct
    pltpu.sync_copy(outv, out_hbm.at[wid])


BR = 8
TC_STEPS = TC_ROWS // BR
TCU = 4


def _tc_body(x_ref, t_ref, o_ref, acc_ref):
    i = pl.program_id(0)
    zero = jnp.zeros((BR, 128), jnp.float32)
    accs = (zero,) * (3 * TCU)

    def lane_body(j, acc):
        acc = list(acc)
        base = j * (TCU * 128)
        for k in range(TCU):
            x = x_ref[:, pl.ds(base + k * 128, 128)]
            t = t_ref[:, pl.ds(base + k * 128, 128)]
            d = x - t
            sq = d * d
            acc[k] = acc[k] + sq * t
            acc[TCU + k] = acc[TCU + k] + sq
            acc[2 * TCU + k] = acc[2 * TCU + k] + t
        return tuple(acc)

    accs = lax.fori_loop(0, COLS // (TCU * 128), lane_body, accs)
    a_bt = functools.reduce(lambda a, b: a + b, accs[0:TCU])
    a_sq = functools.reduce(lambda a, b: a + b, accs[TCU:2 * TCU])
    a_ct = functools.reduce(lambda a, b: a + b, accs[2 * TCU:3 * TCU])

    @pl.when(i == 0)
    def _():
        acc_ref[0] = a_bt
        acc_ref[1] = a_sq
        acc_ref[2] = a_ct

    @pl.when(i > 0)
    def _():
        acc_ref[0] += a_bt
        acc_ref[1] += a_sq
        acc_ref[2] += a_ct

    @pl.when(i == TC_STEPS - 1)
    def _():
        o_ref[0] = jnp.sum(acc_ref[0])
        o_ref[1] = jnp.sum(acc_ref[1])
        o_ref[2] = jnp.sum(acc_ref[2])


_tc_partial = pl.pallas_call(
    _tc_body,
    grid=(TC_STEPS,),
    in_specs=[
        pl.BlockSpec((BR, COLS), lambda i: (SC_ROWS // BR + i, 0)),
        pl.BlockSpec((BR, COLS), lambda i: (SC_ROWS // BR + i, 0)),
    ],
    out_specs=pl.BlockSpec(memory_space=pltpu.SMEM),
    out_shape=jax.ShapeDtypeStruct((3,), jnp.float32),
    scratch_shapes=[pltpu.VMEM((3, BR, 128), jnp.float32)],
    compiler_params=pltpu.CompilerParams(
        dimension_semantics=("arbitrary",),
    ),
)


def kernel(input, target):
    sc_p = _sc_partial(input, target)
    tc_p = _tc_partial(input, target)
    bt = tc_p[0] + jnp.sum(sc_p[:, 0:L])
    sq = tc_p[1] + jnp.sum(sc_p[:, L:2 * L])
    ct = tc_p[2] + jnp.sum(sc_p[:, 2 * L:3 * L])
    beat_count = jnp.maximum(ct, 1.0)
    no_beat_count = jnp.maximum(jnp.float32(N_TOTAL) - ct, 1.0)
    beat_loss = bt / beat_count
    no_beat_loss = (sq - bt) / no_beat_count
    return (no_beat_loss + beat_loss, beat_loss, no_beat_loss)
